# baseline (speedup 1.0000x reference)
import functools

import jax
import jax.numpy as jnp
from jax import lax
from jax.experimental import pallas as pl
from jax.experimental.pallas import tpu as pltpu
from jax.experimental.pallas import tpu_sc as plsc

SEQ = 200
EMB = 64
BPW = 128
SCALE = 8.0

_info = plsc.get_sparse_core_info()
_NC, _NS, _NL = _info.num_cores, _info.num_subcores, _info.num_lanes
_NW = _NC * _NS
_ND = EMB // _NL


def _build(batch: int):
  assert batch == BPW * _NW

  mesh = plsc.VectorSubcoreMesh(core_axis_name="c", subcore_axis_name="s")

  @functools.partial(
      pl.kernel,
      mesh=mesh,
      compiler_params=pltpu.CompilerParams(
          use_tc_tiling_on_sc=False, needs_layout_passes=False),
      out_type=jax.ShapeDtypeStruct(
          (SEQ, EMB // 8, batch // BPW, 8, BPW), jnp.float32),
      scratch_types=[
          pltpu.VMEM((SEQ, BPW), jnp.int32),
          pltpu.VMEM((BPW, BPW), jnp.float32),
          pltpu.VMEM((BPW, BPW), jnp.float32),
          pltpu.VMEM((EMB, BPW), jnp.float32),
          pltpu.VMEM((EMB, BPW), jnp.float32),
          pltpu.VMEM((SEQ, EMB), jnp.float32),
          pltpu.SemaphoreType.DMA,
          pltpu.SemaphoreType.DMA,
          pltpu.SemaphoreType.DMA,
          pltpu.SemaphoreType.DMA,
      ],
  )
  def emb(idx_hbm, table_hbm, pos_hbm, out_hbm,
          idx_v, g0, g1, t0, t1, pos_v, sg0, sg1, so0, so1):
    wid = lax.axis_index("s") * _NC + lax.axis_index("c")
    b0 = wid * BPW
    gbuf = (g0, g1)
    tbuf = (t0, t1)
    sg = (sg0, sg1)
    so = (so0, so1)

    pltpu.sync_copy(idx_hbm.at[:, pl.ds(b0, BPW)], idx_v)
    pltpu.sync_copy(pos_hbm, pos_v)

    lanes = lax.iota(jnp.int32, _NL)
    fidx = [lanes + d * _NL for d in range(_ND)]

    def gather(l, slot):
      return pltpu.make_async_copy(
          table_hbm.at[idx_v.at[l]], gbuf[slot], sg[slot])

    def out_copies(l, slot):
      return [
          pltpu.make_async_copy(
              tbuf[slot].at[pl.ds(f * 8, 8)], out_hbm.at[l, f, wid], so[slot])
          for f in range(EMB // 8)
      ]

    def compute(l, slot):
      g = gbuf[slot]
      t = tbuf[slot]
      pv = [pos_v[l, pl.ds(d * _NL, _NL)] for d in range(_ND)]

      def body(b, c):
        bv = jnp.full((_NL,), 0, jnp.int32) + b
        for d in range(_ND):
          r = g[b, pl.ds(d * _NL, _NL)] * SCALE + pv[d]
          plsc.store_scatter(t, [fidx[d], bv], r)
        return c

      lax.fori_loop(0, BPW, body, 0, unroll=2)

    def step(l, slot):
      gather(l, slot).wait()

      @pl.when(l + 1 < SEQ)
      def _():
        gather(l + 1, 1 - slot).start()

      @pl.when(l >= 2)
      def _():
        for c in out_copies(l - 2, slot):
          c.wait()

      compute(l, slot)
      for c in out_copies(l, slot):
        c.start()

    gather(0, 0).start()

    def pair(p, c):
      l = 2 * p
      step(l, 0)
      step(l + 1, 1)
      return c

    lax.fori_loop(0, SEQ // 2, pair, 0)

    for c in out_copies(SEQ - 2, 0):
      c.wait()
    for c in out_copies(SEQ - 1, 1):
      c.wait()

  return emb


def kernel(inputs, token_table, position_table):
  batch = inputs.shape[0]
  tabp = jnp.pad(token_table, ((0, 0), (0, BPW - EMB)))
  emb = _build(batch)
  out5 = emb(inputs.T, tabp, position_table)
  return out5.transpose(2, 4, 0, 1, 3).reshape(batch, SEQ, EMB)
